# Initial kernel scaffold; baseline (speedup 1.0000x reference)
import jax
import jax.numpy as jnp
from jax.experimental import pallas as pl


def kernel(x_parts, selected_experts, W12, b12, W1, b1, W2, b2):
    raise NotImplementedError("write your pallas kernel here")


# speedup vs baseline: 2.6363x; 2.6363x over previous
import functools

import jax
import jax.numpy as jnp
from jax import lax
from jax.experimental import pallas as pl
from jax.experimental.pallas import tpu as pltpu
from jax.experimental.pallas import tpu_sc as plsc

_TILE = 256


def _routing(sel, n_experts, tile, n_tiles):
    t_tok = sel.shape[0]
    order = jnp.argsort(sel, stable=True).astype(jnp.int32)
    sel_sorted = jnp.take(sel, order)
    counts = jnp.bincount(sel, length=n_experts).astype(jnp.int32)
    padded = ((counts + tile - 1) // tile) * tile
    ends_c = jnp.cumsum(counts)
    starts_c = ends_c - counts
    ends_p = jnp.cumsum(padded)
    starts_p = ends_p - padded
    rank = jnp.arange(t_tok, dtype=jnp.int32) - jnp.take(starts_c, sel_sorted)
    posj = jnp.take(starts_p, sel_sorted) + rank
    pt = n_tiles * tile
    src = jnp.zeros((pt,), jnp.int32).at[posj].set(order)
    pos = jnp.zeros((t_tok,), jnp.int32).at[order].set(posj)
    tile_start = jnp.arange(n_tiles, dtype=jnp.int32) * tile
    eids_raw = jnp.searchsorted(ends_p, tile_start, side="right").astype(jnp.int32)
    total = ends_p[-1]
    used = (tile_start < total).astype(jnp.int32)
    last_eid = jnp.take(eids_raw, total // tile - 1)
    eids = jnp.where(used == 1, eids_raw, last_eid)
    return src, pos, eids, used


def _sc_gather_rows(table, idx):
    _, d = table.shape
    b = idx.shape[0]
    info = plsc.get_sparse_core_info()
    nw = info.num_cores * info.num_subcores
    b_per_w = b // nw
    ch = 64 if b_per_w % 64 == 0 else b_per_w
    nch = b_per_w // ch
    mesh = plsc.VectorSubcoreMesh(core_axis_name="c", subcore_axis_name="s")

    @functools.partial(
        pl.kernel,
        mesh=mesh,
        out_type=jax.ShapeDtypeStruct((b, d), table.dtype),
        scratch_types=[
            pltpu.VMEM((ch,), jnp.int32),
            pltpu.VMEM((ch, d), table.dtype),
            pltpu.SemaphoreType.DMA,
        ],
    )
    def gather_k(table_hbm, idx_hbm, out_hbm, idx_v, rows_v, sem):
        wid = lax.axis_index("s") * info.num_cores + lax.axis_index("c")
        base = wid * b_per_w
        for c in range(nch):
            off = base + c * ch
            pltpu.sync_copy(idx_hbm.at[pl.ds(off, ch)], idx_v)
            pltpu.async_copy(table_hbm.at[idx_v], rows_v, sem).wait()
            pltpu.sync_copy(rows_v, out_hbm.at[pl.ds(off, ch)])

    return gather_k(table, idx)


def _swiglu_body(eids_ref, used_ref, x_ref, w_ref, b_ref, s_ref):
    t = pl.program_id(0)

    @pl.when(used_ref[t] == 1)
    def _():
        e = eids_ref[t]
        x = x_ref[...].astype(jnp.bfloat16)
        w = w_ref[0].astype(jnp.bfloat16)
        h = jnp.dot(x, w, preferred_element_type=jnp.float32)
        h = h + b_ref[pl.ds(e, 1), :]
        half = h.shape[1] // 2
        a = h[:, :half]
        g = h[:, half:]
        s_ref[...] = ((a / (1.0 + jnp.exp(-a))) * g).astype(jnp.bfloat16)


def _mlp_body(eids_ref, used_ref, s_ref, w1_ref, b1_ref, w2_ref, b2_ref, o_ref,
              acc_ref):
    jc = pl.program_id(0)
    t = pl.program_id(1)
    nj = pl.num_programs(0)

    @pl.when(used_ref[t] == 1)
    def _():
        e = eids_ref[t]
        s = s_ref[...]
        tile = s.shape[0]
        w1 = w1_ref[0].astype(jnp.bfloat16)
        h1 = jnp.dot(s, w1, preferred_element_type=jnp.float32)
        h1 = jnp.maximum(h1 + b1_ref[pl.ds(e, 1), :], 0.0).astype(jnp.bfloat16)
        w2 = w2_ref[0].astype(jnp.bfloat16)
        part = jnp.dot(h1, w2, preferred_element_type=jnp.float32)
        sl = pl.ds(t * tile, tile)

        @pl.when(jc == 0)
        def _():
            acc_ref[sl, :] = part + b2_ref[pl.ds(e, 1), :]

        @pl.when(jc > 0)
        def _():
            acc_ref[sl, :] = acc_ref[sl, :] + part

        @pl.when(jc == nj - 1)
        def _():
            o_ref[...] = acc_ref[sl, :]


def kernel(x_parts, selected_experts, W12, b12, W1, b1, W2, b2):
    pp, nn, kk, f = x_parts.shape
    e_num, _, h2 = W12.shape
    h = h2 // 2
    out_d = W2.shape[-1]
    t_tok = pp * nn * kk
    tile = _TILE
    n_tiles = t_tok // tile + e_num
    pt = n_tiles * tile

    xf = x_parts.reshape(t_tok, f)
    sel = selected_experts.reshape(t_tok).astype(jnp.int32)
    src, pos, eids, used = _routing(sel, e_num, tile, n_tiles)

    x_sorted = _sc_gather_rows(xf, src)

    s = pl.pallas_call(
        _swiglu_body,
        grid_spec=pltpu.PrefetchScalarGridSpec(
            num_scalar_prefetch=2,
            grid=(n_tiles,),
            in_specs=[
                pl.BlockSpec((tile, f), lambda t, eids, used: (t, 0)),
                pl.BlockSpec((1, f, h2), lambda t, eids, used: (eids[t], 0, 0)),
                pl.BlockSpec((e_num, h2), lambda t, eids, used: (0, 0)),
            ],
            out_specs=pl.BlockSpec((tile, h), lambda t, eids, used: (t, 0)),
        ),
        out_shape=jax.ShapeDtypeStruct((pt, h), jnp.bfloat16),
        compiler_params=pltpu.CompilerParams(
            dimension_semantics=("arbitrary",),
        ),
    )(eids, used, x_sorted, W12, b12)

    hb = 768
    nj = h // hb
    out_sorted = pl.pallas_call(
        _mlp_body,
        grid_spec=pltpu.PrefetchScalarGridSpec(
            num_scalar_prefetch=2,
            grid=(nj, n_tiles),
            in_specs=[
                pl.BlockSpec((tile, h), lambda jc, t, eids, used: (t, 0)),
                pl.BlockSpec((1, h, hb), lambda jc, t, eids, used: (eids[t], 0, jc)),
                pl.BlockSpec((e_num, hb), lambda jc, t, eids, used: (0, jc)),
                pl.BlockSpec((1, hb, out_d), lambda jc, t, eids, used: (eids[t], jc, 0)),
                pl.BlockSpec((e_num, out_d), lambda jc, t, eids, used: (0, 0)),
            ],
            out_specs=pl.BlockSpec((tile, out_d), lambda jc, t, eids, used: (t, 0)),
            scratch_shapes=[pltpu.VMEM((pt, out_d), jnp.float32)],
        ),
        out_shape=jax.ShapeDtypeStruct((pt, out_d), jnp.float32),
        compiler_params=pltpu.CompilerParams(
            dimension_semantics=("arbitrary", "arbitrary"),
        ),
    )(eids, used, s, W1, b1, W2, b2)

    out_f = _sc_gather_rows(out_sorted, pos)
    return out_f.reshape(pp, nn, kk, out_d)
